# baseline (speedup 1.0000x reference)
import functools

import jax
import jax.numpy as jnp
from jax import lax
from jax.experimental import pallas as pl
from jax.experimental.pallas import tpu as pltpu
from jax.experimental.pallas import tpu_sc as plsc

D_MODEL_K = 2048
NUM_TILES_K = 4
D_FF_K = D_MODEL_K * 4
TILE_K = D_FF_K // NUM_TILES_K
N_K = 2 * 4096
ROW_BLK = 512
TB = 512
TBM = 256
G_K = N_K // TBM + NUM_TILES_K - 1
NC, NS = 2, 16
NW = NC * NS
TPW = N_K // NW
CH = 32


def _prep_body(up_ref, down_ref, upsign_ref, downsign_ref, s_ref):
    g = pl.program_id(0)
    usign = jnp.sign(up_ref[...])
    upsign_ref[...] = usign.astype(jnp.bfloat16)
    downsign_ref[...] = jnp.sign(down_ref[...]).astype(jnp.bfloat16)
    blk_sum = jnp.sum(usign, axis=0, keepdims=True)[None]
    @pl.when(g % (TILE_K // ROW_BLK) == 0)
    def _init():
        s_ref[...] = jnp.zeros_like(s_ref)
    s_ref[...] += blk_sum


def _router_body(x_ref, s_ref, winner_ref, gate_ref, x16_ref):
    xb = x_ref[...]
    m = s_ref[...] * (1.0 / TILE_K)
    denom = jnp.sqrt(jnp.sum(m * m, axis=-1, keepdims=True)) + 1e-8
    sigs = m / denom
    xb16 = xb.astype(jnp.bfloat16)
    x16_ref[...] = xb16
    scores = lax.dot_general(xb16, sigs.astype(jnp.bfloat16),
                             (((1,), (1,)), ((), ())),
                             preferred_element_type=jnp.float32)
    winner = jnp.argmax(scores, axis=-1).astype(jnp.int32)
    gate_ref[...] = (
        winner[:, None] == lax.broadcasted_iota(jnp.int32, (1, NUM_TILES_K), 1)
    ).astype(jnp.float32)
    winner_ref[...] = winner


def _cumsum_axis(x, axis):
    size = x.shape[axis]
    sh = 1
    while sh < size:
        if axis == 1:
            shifted = jnp.concatenate(
                [jnp.zeros((x.shape[0], sh), x.dtype), x[:, :-sh]], axis=1)
        else:
            shifted = jnp.concatenate(
                [jnp.zeros((sh, x.shape[1]), x.dtype), x[:-sh, :]], axis=0)
        x = x + shifted
        sh *= 2
    return x


def _dispatch_body(w_ref, dest_ref, btile_ref):
    w = w_ref[...]
    within = jnp.zeros_like(w)
    counts = []
    for t in range(NUM_TILES_K):
        mt = (w == t).astype(jnp.int32)
        lane_cum = _cumsum_axis(mt, axis=1)
        row_tot = lane_cum[:, -1:]
        row_cum = _cumsum_axis(row_tot, axis=0)
        excl = (row_cum - row_tot) + (lane_cum - mt)
        within = within + mt * excl
        counts.append(row_cum[-1:, :])
    nb = [(c + (TBM - 1)) // TBM for c in counts]
    seg = jnp.zeros_like(counts[0])
    dest = within
    cum = jnp.zeros_like(counts[0])
    iota_g = lax.broadcasted_iota(jnp.int32, (1, G_K), 1)
    btile = jnp.zeros((1, G_K), dtype=jnp.int32)
    for t in range(NUM_TILES_K):
        dest = dest + (w == t).astype(jnp.int32) * (seg * TBM)
        cum = cum + nb[t]
        if t < NUM_TILES_K - 1:
            btile = btile + (iota_g >= cum).astype(jnp.int32)
        seg = cum
    dest_ref[...] = dest
    btile_ref[...] = btile


def _sc_scatter_body(x_hbm, dest_hbm, xbuf_hbm, idx_v, rows_v, sem):
    wid = lax.axis_index("s") * NC + lax.axis_index("c")
    base = wid * TPW
    for ci in range(TPW // CH):
        off = base + ci * CH
        pltpu.sync_copy(dest_hbm.at[pl.ds(off, CH)], idx_v)
        pltpu.sync_copy(x_hbm.at[pl.ds(off, CH)], rows_v)
        pltpu.async_copy(rows_v, xbuf_hbm.at[idx_v], sem).wait()


def _sc_gather_body(ybuf_hbm, dest_hbm, out_hbm, idx_v, rows_v, sem):
    wid = lax.axis_index("s") * NC + lax.axis_index("c")
    base = wid * TPW
    for ci in range(TPW // CH):
        off = base + ci * CH
        pltpu.sync_copy(dest_hbm.at[pl.ds(off, CH)], idx_v)
        pltpu.async_copy(ybuf_hbm.at[idx_v], rows_v, sem).wait()
        pltpu.sync_copy(rows_v, out_hbm.at[pl.ds(off, CH)])


@functools.lru_cache(maxsize=None)
def _sc_kernels():
    mesh = plsc.VectorSubcoreMesh(core_axis_name="c", subcore_axis_name="s",
                                  num_cores=NC, num_subcores=NS)
    scratch = [
        pltpu.VMEM((CH,), jnp.int32),
        pltpu.VMEM((CH, D_MODEL_K // 2), jnp.int32),
        pltpu.SemaphoreType.DMA,
    ]
    scatter = pl.kernel(
        _sc_scatter_body,
        out_type=jax.ShapeDtypeStruct((G_K * TBM, D_MODEL_K // 2), jnp.int32),
        mesh=mesh, scratch_types=scratch)
    gather = pl.kernel(
        _sc_gather_body,
        out_type=jax.ShapeDtypeStruct((N_K, D_MODEL_K // 2), jnp.int32),
        mesh=mesh, scratch_types=scratch)
    return scatter, gather


def _ffn_body(btile_ref, xbuf_ref, upsign_ref, downsign_ref, upsc_ref,
              downsc_ref, ybuf_ref):
    xb16 = xbuf_ref[...]
    h = lax.dot_general(xb16, upsign_ref[...], (((1,), (1,)), ((), ())),
                        preferred_element_type=jnp.float32)
    h = jnp.maximum(h * upsc_ref[...], 0.0)
    y = lax.dot_general(h.astype(jnp.bfloat16), downsign_ref[...],
                        (((1,), (1,)), ((), ())),
                        preferred_element_type=jnp.float32)
    ybuf_ref[...] = (y * downsc_ref[...]).astype(jnp.bfloat16)


@jax.jit
def _run(x, up_w, up_scales, down_w, down_scales):
    b, t, c = x.shape
    n = b * t
    xf = x.reshape(n, c)

    upsign, downsign, s = pl.pallas_call(
        _prep_body,
        grid=(D_FF_K // ROW_BLK,),
        in_specs=[
            pl.BlockSpec((ROW_BLK, D_MODEL_K), lambda g: (g, 0)),
            pl.BlockSpec((D_MODEL_K, ROW_BLK), lambda g: (0, g)),
        ],
        out_specs=[
            pl.BlockSpec((ROW_BLK, D_MODEL_K), lambda g: (g, 0)),
            pl.BlockSpec((D_MODEL_K, ROW_BLK), lambda g: (0, g)),
            pl.BlockSpec((1, 1, D_MODEL_K), lambda g: (g // (TILE_K // ROW_BLK), 0, 0)),
        ],
        out_shape=[
            jax.ShapeDtypeStruct((D_FF_K, D_MODEL_K), jnp.bfloat16),
            jax.ShapeDtypeStruct((D_MODEL_K, D_FF_K), jnp.bfloat16),
            jax.ShapeDtypeStruct((NUM_TILES_K, 1, D_MODEL_K), jnp.float32),
        ],
    )(up_w, down_w)
    s = s.reshape(NUM_TILES_K, D_MODEL_K)

    winner, gate, x16 = pl.pallas_call(
        _router_body,
        grid=(n // TB,),
        in_specs=[
            pl.BlockSpec((TB, D_MODEL_K), lambda g: (g, 0)),
            pl.BlockSpec((NUM_TILES_K, D_MODEL_K), lambda g: (0, 0)),
        ],
        out_specs=[
            pl.BlockSpec((TB,), lambda g: (g,)),
            pl.BlockSpec((TB, NUM_TILES_K), lambda g: (g, 0)),
            pl.BlockSpec((TB, D_MODEL_K), lambda g: (g, 0)),
        ],
        out_shape=[
            jax.ShapeDtypeStruct((n,), jnp.int32),
            jax.ShapeDtypeStruct((n, NUM_TILES_K), jnp.float32),
            jax.ShapeDtypeStruct((n, D_MODEL_K), jnp.bfloat16),
        ],
    )(xf, s)

    dest, btile = pl.pallas_call(
        _dispatch_body,
        out_shape=[
            jax.ShapeDtypeStruct((n // 128, 128), jnp.int32),
            jax.ShapeDtypeStruct((1, G_K), jnp.int32),
        ],
    )(winner.reshape(n // 128, 128))
    dest = dest.reshape(n)
    btile = btile.reshape(G_K)

    sc_scatter, sc_gather = _sc_kernels()
    x16_i = lax.bitcast_convert_type(
        x16.reshape(n, D_MODEL_K // 2, 2), jnp.int32)
    xbuf_i = sc_scatter(x16_i, dest)
    xbuf = lax.bitcast_convert_type(xbuf_i, jnp.bfloat16).reshape(
        G_K * TBM, D_MODEL_K)

    ybuf = pl.pallas_call(
        _ffn_body,
        grid_spec=pltpu.PrefetchScalarGridSpec(
            num_scalar_prefetch=1,
            grid=(G_K,),
            in_specs=[
                pl.BlockSpec((TBM, D_MODEL_K), lambda g, bt: (g, 0)),
                pl.BlockSpec((TILE_K, D_MODEL_K), lambda g, bt: (bt[g], 0)),
                pl.BlockSpec((D_MODEL_K, TILE_K), lambda g, bt: (0, bt[g])),
                pl.BlockSpec((1, TILE_K), lambda g, bt: (0, bt[g])),
                pl.BlockSpec((1, D_MODEL_K), lambda g, bt: (0, 0)),
            ],
            out_specs=pl.BlockSpec((TBM, D_MODEL_K), lambda g, bt: (g, 0)),
        ),
        out_shape=jax.ShapeDtypeStruct((G_K * TBM, D_MODEL_K), jnp.bfloat16),
    )(btile, xbuf, upsign, downsign, up_scales.reshape(1, D_FF_K),
      down_scales.reshape(1, D_MODEL_K))

    ybuf_i = lax.bitcast_convert_type(
        ybuf.reshape(G_K * TBM, D_MODEL_K // 2, 2), jnp.int32)
    out_i = sc_gather(ybuf_i, dest)
    out = lax.bitcast_convert_type(out_i, jnp.bfloat16).reshape(n, D_MODEL_K)

    return out.astype(jnp.float32).reshape(b, t, c), gate.reshape(b, t, NUM_TILES_K)


def kernel(x, up_w, up_scales, down_w, down_scales):
    return _run(x, up_w, up_scales, down_w, down_scales)


# speedup vs baseline: 4.6209x; 4.6209x over previous
import functools

import jax
import jax.numpy as jnp
from jax import lax
from jax.experimental import pallas as pl
from jax.experimental.pallas import tpu as pltpu
from jax.experimental.pallas import tpu_sc as plsc

D_MODEL_K = 2048
NUM_TILES_K = 4
D_FF_K = D_MODEL_K * 4
TILE_K = D_FF_K // NUM_TILES_K
N_K = 2 * 4096
ROW_BLK = 512
TB = 512
TBM = 256
G_K = N_K // TBM + NUM_TILES_K - 1
NC, NS = 2, 16
NW = NC * NS
TPW = N_K // NW
CH = 32


def _prep_up_body(up_ref, upsign_ref, s_ref):
    g = pl.program_id(0)
    usign = jnp.sign(up_ref[...])
    upsign_ref[...] = usign.astype(jnp.bfloat16)
    blk_sum = jnp.sum(usign, axis=0, keepdims=True)[None]
    @pl.when(g % (TILE_K // ROW_BLK) == 0)
    def _init():
        s_ref[...] = jnp.zeros_like(s_ref)
    s_ref[...] += blk_sum


def _prep_down_body(down_ref, downsign_ref):
    downsign_ref[...] = jnp.sign(down_ref[...]).astype(jnp.bfloat16)


def _router_body(x_ref, s_ref, winner_ref, gate_ref):
    xb = x_ref[...]
    m = s_ref[...] * (1.0 / TILE_K)
    denom = jnp.sqrt(jnp.sum(m * m, axis=-1, keepdims=True)) + 1e-8
    sigs = m / denom
    scores = lax.dot_general(xb.astype(jnp.bfloat16), sigs.astype(jnp.bfloat16),
                             (((1,), (1,)), ((), ())),
                             preferred_element_type=jnp.float32)
    winner = jnp.argmax(scores, axis=-1).astype(jnp.int32)
    gate_ref[...] = (
        winner[:, None] == lax.broadcasted_iota(jnp.int32, (1, NUM_TILES_K), 1)
    ).astype(jnp.float32)
    winner_ref[...] = winner


def _cumsum_axis(x, axis):
    size = x.shape[axis]
    sh = 1
    while sh < size:
        if axis == 1:
            shifted = jnp.concatenate(
                [jnp.zeros((x.shape[0], sh), x.dtype), x[:, :-sh]], axis=1)
        else:
            shifted = jnp.concatenate(
                [jnp.zeros((sh, x.shape[1]), x.dtype), x[:-sh, :]], axis=0)
        x = x + shifted
        sh *= 2
    return x


def _dispatch_body(w_ref, dest_ref, btile_ref):
    w = w_ref[...]
    within = jnp.zeros_like(w)
    counts = []
    for t in range(NUM_TILES_K):
        mt = (w == t).astype(jnp.int32)
        lane_cum = _cumsum_axis(mt, axis=1)
        row_tot = lane_cum[:, -1:]
        row_cum = _cumsum_axis(row_tot, axis=0)
        excl = (row_cum - row_tot) + (lane_cum - mt)
        within = within + mt * excl
        counts.append(row_cum[-1:, :])
    nb = [(c + (TBM - 1)) // TBM for c in counts]
    seg = jnp.zeros_like(counts[0])
    dest = within
    cum = jnp.zeros_like(counts[0])
    iota_g = lax.broadcasted_iota(jnp.int32, (1, G_K), 1)
    btile = jnp.zeros((1, G_K), dtype=jnp.int32)
    for t in range(NUM_TILES_K):
        dest = dest + (w == t).astype(jnp.int32) * (seg * TBM)
        cum = cum + nb[t]
        if t < NUM_TILES_K - 1:
            btile = btile + (iota_g >= cum).astype(jnp.int32)
        seg = cum
    dest_ref[...] = dest
    btile_ref[...] = btile


def _sc_scatter_body(x_hbm, dest_hbm, xbuf_hbm, idx_v, rows_v, sem):
    wid = lax.axis_index("s") * NC + lax.axis_index("c")
    base = wid * TPW
    for ci in range(TPW // CH):
        off = base + ci * CH
        pltpu.sync_copy(dest_hbm.at[pl.ds(off, CH)], idx_v)
        pltpu.sync_copy(x_hbm.at[pl.ds(off, CH)], rows_v)
        pltpu.async_copy(rows_v, xbuf_hbm.at[idx_v], sem).wait()


def _sc_gather_body(ybuf_hbm, dest_hbm, out_hbm, idx_v, rows_v, sem):
    wid = lax.axis_index("s") * NC + lax.axis_index("c")
    base = wid * TPW
    for ci in range(TPW // CH):
        off = base + ci * CH
        pltpu.sync_copy(dest_hbm.at[pl.ds(off, CH)], idx_v)
        pltpu.async_copy(ybuf_hbm.at[idx_v], rows_v, sem).wait()
        pltpu.sync_copy(rows_v, out_hbm.at[pl.ds(off, CH)])


@functools.lru_cache(maxsize=None)
def _sc_kernels():
    mesh = plsc.VectorSubcoreMesh(core_axis_name="c", subcore_axis_name="s",
                                  num_cores=NC, num_subcores=NS)
    scratch = [
        pltpu.VMEM((CH,), jnp.int32),
        pltpu.VMEM((CH, D_MODEL_K), jnp.float32),
        pltpu.SemaphoreType.DMA,
    ]
    scatter = pl.kernel(
        _sc_scatter_body,
        out_type=jax.ShapeDtypeStruct((G_K * TBM, D_MODEL_K), jnp.float32),
        mesh=mesh, scratch_types=scratch)
    gather = pl.kernel(
        _sc_gather_body,
        out_type=jax.ShapeDtypeStruct((N_K, D_MODEL_K), jnp.float32),
        mesh=mesh, scratch_types=scratch)
    return scatter, gather


def _ffn_body(btile_ref, xbuf_ref, upsign_ref, downsign_ref, upsc_ref,
              downsc_ref, ybuf_ref):
    xb16 = xbuf_ref[...].astype(jnp.bfloat16)
    h = lax.dot_general(xb16, upsign_ref[...], (((1,), (1,)), ((), ())),
                        preferred_element_type=jnp.float32)
    h = jnp.maximum(h * upsc_ref[...], 0.0)
    y = lax.dot_general(h.astype(jnp.bfloat16), downsign_ref[...],
                        (((1,), (1,)), ((), ())),
                        preferred_element_type=jnp.float32)
    ybuf_ref[...] = y * downsc_ref[...]


@jax.jit
def _run(x, up_w, up_scales, down_w, down_scales):
    b, t, c = x.shape
    n = b * t
    xf = x.reshape(n, c)

    upsign, s = pl.pallas_call(
        _prep_up_body,
        grid=(D_FF_K // ROW_BLK,),
        in_specs=[
            pl.BlockSpec((ROW_BLK, D_MODEL_K), lambda g: (g, 0)),
        ],
        out_specs=[
            pl.BlockSpec((ROW_BLK, D_MODEL_K), lambda g: (g, 0)),
            pl.BlockSpec((1, 1, D_MODEL_K), lambda g: (g // (TILE_K // ROW_BLK), 0, 0)),
        ],
        out_shape=[
            jax.ShapeDtypeStruct((D_FF_K, D_MODEL_K), jnp.bfloat16),
            jax.ShapeDtypeStruct((NUM_TILES_K, 1, D_MODEL_K), jnp.float32),
        ],
    )(up_w)
    s = s.reshape(NUM_TILES_K, D_MODEL_K)

    downsign = pl.pallas_call(
        _prep_down_body,
        grid=(D_FF_K // ROW_BLK,),
        in_specs=[pl.BlockSpec((D_MODEL_K, ROW_BLK), lambda g: (0, g))],
        out_specs=pl.BlockSpec((D_MODEL_K, ROW_BLK), lambda g: (0, g)),
        out_shape=jax.ShapeDtypeStruct((D_MODEL_K, D_FF_K), jnp.bfloat16),
    )(down_w)

    winner, gate = pl.pallas_call(
        _router_body,
        grid=(n // TB,),
        in_specs=[
            pl.BlockSpec((TB, D_MODEL_K), lambda g: (g, 0)),
            pl.BlockSpec((NUM_TILES_K, D_MODEL_K), lambda g: (0, 0)),
        ],
        out_specs=[
            pl.BlockSpec((TB,), lambda g: (g,)),
            pl.BlockSpec((TB, NUM_TILES_K), lambda g: (g, 0)),
        ],
        out_shape=[
            jax.ShapeDtypeStruct((n,), jnp.int32),
            jax.ShapeDtypeStruct((n, NUM_TILES_K), jnp.float32),
        ],
    )(xf, s)

    dest, btile = pl.pallas_call(
        _dispatch_body,
        out_shape=[
            jax.ShapeDtypeStruct((n // 128, 128), jnp.int32),
            jax.ShapeDtypeStruct((1, G_K), jnp.int32),
        ],
    )(winner.reshape(n // 128, 128))
    dest = dest.reshape(n)
    btile = btile.reshape(G_K)

    sc_scatter, sc_gather = _sc_kernels()
    xbuf = sc_scatter(xf, dest)

    ybuf = pl.pallas_call(
        _ffn_body,
        grid_spec=pltpu.PrefetchScalarGridSpec(
            num_scalar_prefetch=1,
            grid=(G_K,),
            in_specs=[
                pl.BlockSpec((TBM, D_MODEL_K), lambda g, bt: (g, 0)),
                pl.BlockSpec((TILE_K, D_MODEL_K), lambda g, bt: (bt[g], 0)),
                pl.BlockSpec((D_MODEL_K, TILE_K), lambda g, bt: (0, bt[g])),
                pl.BlockSpec((1, TILE_K), lambda g, bt: (0, bt[g])),
                pl.BlockSpec((1, D_MODEL_K), lambda g, bt: (0, 0)),
            ],
            out_specs=pl.BlockSpec((TBM, D_MODEL_K), lambda g, bt: (g, 0)),
        ),
        out_shape=jax.ShapeDtypeStruct((G_K * TBM, D_MODEL_K), jnp.float32),
    )(btile, xbuf, upsign, downsign, up_scales.reshape(1, D_FF_K),
      down_scales.reshape(1, D_MODEL_K))

    out = sc_gather(ybuf, dest)

    return out.reshape(b, t, c), gate.reshape(b, t, NUM_TILES_K)


def kernel(x, up_w, up_scales, down_w, down_scales):
    return _run(x, up_w, up_scales, down_w, down_scales)


# speedup vs baseline: 4.8204x; 1.0432x over previous
import functools

import jax
import jax.numpy as jnp
from jax import lax
from jax.experimental import pallas as pl
from jax.experimental.pallas import tpu as pltpu
from jax.experimental.pallas import tpu_sc as plsc

D_MODEL_K = 2048
NUM_TILES_K = 4
D_FF_K = D_MODEL_K * 4
TILE_K = D_FF_K // NUM_TILES_K
N_K = 2 * 4096
ROW_BLK = 512
TB = 512
TBM = 256
G_K = N_K // TBM + NUM_TILES_K - 1
NC, NS = 2, 16
NW = NC * NS
TPW = N_K // NW
CH = 32


def _prep_up_body(up_ref, upsign_ref, s_ref):
    g = pl.program_id(0)
    usign = jnp.sign(up_ref[...])
    upsign_ref[...] = usign.astype(jnp.bfloat16)
    blk_sum = jnp.sum(usign, axis=0, keepdims=True)[None]
    @pl.when(g % (TILE_K // ROW_BLK) == 0)
    def _init():
        s_ref[...] = jnp.zeros_like(s_ref)
    s_ref[...] += blk_sum


def _prep_down_body(down_ref, downsign_ref):
    downsign_ref[...] = jnp.sign(down_ref[...]).astype(jnp.bfloat16)


def _pack_bf16_pair(lo16, hi16):
    lo_u = lax.convert_element_type(
        lax.bitcast_convert_type(lo16, jnp.uint16), jnp.uint32)
    hi_u = lax.convert_element_type(
        lax.bitcast_convert_type(hi16, jnp.uint16), jnp.uint32)
    return lax.bitcast_convert_type(lo_u | (hi_u << 16), jnp.int32)


def _unpack_bf16_pair(packed):
    pu = lax.bitcast_convert_type(packed, jnp.uint32)
    lo = lax.bitcast_convert_type(
        lax.convert_element_type(pu & 0xFFFF, jnp.uint16), jnp.bfloat16)
    hi = lax.bitcast_convert_type(
        lax.convert_element_type(pu >> 16, jnp.uint16), jnp.bfloat16)
    return lo, hi


def _router_body(x_ref, s_ref, winner_ref, gate_ref, xp_ref):
    xb = x_ref[...]
    m = s_ref[...] * (1.0 / TILE_K)
    denom = jnp.sqrt(jnp.sum(m * m, axis=-1, keepdims=True)) + 1e-8
    sigs = m / denom
    xb16 = xb.astype(jnp.bfloat16)
    xp_ref[...] = _pack_bf16_pair(xb16[:, :D_MODEL_K // 2],
                                  xb16[:, D_MODEL_K // 2:])
    scores = lax.dot_general(xb16, sigs.astype(jnp.bfloat16),
                             (((1,), (1,)), ((), ())),
                             preferred_element_type=jnp.float32)
    winner = jnp.argmax(scores, axis=-1).astype(jnp.int32)
    gate_ref[...] = (
        winner[:, None] == lax.broadcasted_iota(jnp.int32, (1, NUM_TILES_K), 1)
    ).astype(jnp.float32)
    winner_ref[...] = winner


def _cumsum_axis(x, axis):
    size = x.shape[axis]
    sh = 1
    while sh < size:
        if axis == 1:
            shifted = jnp.concatenate(
                [jnp.zeros((x.shape[0], sh), x.dtype), x[:, :-sh]], axis=1)
        else:
            shifted = jnp.concatenate(
                [jnp.zeros((sh, x.shape[1]), x.dtype), x[:-sh, :]], axis=0)
        x = x + shifted
        sh *= 2
    return x


def _dispatch_body(w_ref, dest_ref, btile_ref):
    w = w_ref[...]
    within = jnp.zeros_like(w)
    counts = []
    for t in range(NUM_TILES_K):
        mt = (w == t).astype(jnp.int32)
        lane_cum = _cumsum_axis(mt, axis=1)
        row_tot = lane_cum[:, -1:]
        row_cum = _cumsum_axis(row_tot, axis=0)
        excl = (row_cum - row_tot) + (lane_cum - mt)
        within = within + mt * excl
        counts.append(row_cum[-1:, :])
    nb = [(c + (TBM - 1)) // TBM for c in counts]
    seg = jnp.zeros_like(counts[0])
    dest = within
    cum = jnp.zeros_like(counts[0])
    iota_g = lax.broadcasted_iota(jnp.int32, (1, G_K), 1)
    btile = jnp.zeros((1, G_K), dtype=jnp.int32)
    for t in range(NUM_TILES_K):
        dest = dest + (w == t).astype(jnp.int32) * (seg * TBM)
        cum = cum + nb[t]
        if t < NUM_TILES_K - 1:
            btile = btile + (iota_g >= cum).astype(jnp.int32)
        seg = cum
    dest_ref[...] = dest
    btile_ref[...] = btile


def _sc_scatter_body(x_hbm, dest_hbm, xbuf_hbm, idx_v, rows_v, sem):
    wid = lax.axis_index("s") * NC + lax.axis_index("c")
    base = wid * TPW
    for ci in range(TPW // CH):
        off = base + ci * CH
        pltpu.sync_copy(dest_hbm.at[pl.ds(off, CH)], idx_v)
        pltpu.sync_copy(x_hbm.at[pl.ds(off, CH)], rows_v)
        pltpu.async_copy(rows_v, xbuf_hbm.at[idx_v], sem).wait()


def _sc_gather_body(ybuf_hbm, dest_hbm, out_hbm, idx_v, rows_v, sem):
    wid = lax.axis_index("s") * NC + lax.axis_index("c")
    base = wid * TPW
    for ci in range(TPW // CH):
        off = base + ci * CH
        pltpu.sync_copy(dest_hbm.at[pl.ds(off, CH)], idx_v)
        pltpu.async_copy(ybuf_hbm.at[idx_v], rows_v, sem).wait()
        pltpu.sync_copy(rows_v, out_hbm.at[pl.ds(off, CH)])


@functools.lru_cache(maxsize=None)
def _sc_kernels():
    mesh = plsc.VectorSubcoreMesh(core_axis_name="c", subcore_axis_name="s",
                                  num_cores=NC, num_subcores=NS)
    scatter = pl.kernel(
        _sc_scatter_body,
        out_type=jax.ShapeDtypeStruct((G_K * TBM, D_MODEL_K // 2), jnp.int32),
        mesh=mesh, scratch_types=[
            pltpu.VMEM((CH,), jnp.int32),
            pltpu.VMEM((CH, D_MODEL_K // 2), jnp.int32),
            pltpu.SemaphoreType.DMA,
        ])
    gather = pl.kernel(
        _sc_gather_body,
        out_type=jax.ShapeDtypeStruct((N_K, D_MODEL_K), jnp.float32),
        mesh=mesh, scratch_types=[
            pltpu.VMEM((CH,), jnp.int32),
            pltpu.VMEM((CH, D_MODEL_K), jnp.float32),
            pltpu.SemaphoreType.DMA,
        ])
    return scatter, gather


def _ffn_body(btile_ref, xbuf_ref, upsign_ref, downsign_ref, upsc_ref,
              downsc_ref, ybuf_ref):
    lo, hi = _unpack_bf16_pair(xbuf_ref[...])
    xb16 = jnp.concatenate([lo, hi], axis=1)
    h = lax.dot_general(xb16, upsign_ref[...], (((1,), (1,)), ((), ())),
                        preferred_element_type=jnp.float32)
    h = jnp.maximum(h * upsc_ref[...], 0.0)
    y = lax.dot_general(h.astype(jnp.bfloat16), downsign_ref[...],
                        (((1,), (1,)), ((), ())),
                        preferred_element_type=jnp.float32)
    ybuf_ref[...] = y * downsc_ref[...]


@jax.jit
def _run(x, up_w, up_scales, down_w, down_scales):
    b, t, c = x.shape
    n = b * t
    xf = x.reshape(n, c)

    upsign, s = pl.pallas_call(
        _prep_up_body,
        grid=(D_FF_K // ROW_BLK,),
        in_specs=[
            pl.BlockSpec((ROW_BLK, D_MODEL_K), lambda g: (g, 0)),
        ],
        out_specs=[
            pl.BlockSpec((ROW_BLK, D_MODEL_K), lambda g: (g, 0)),
            pl.BlockSpec((1, 1, D_MODEL_K), lambda g: (g // (TILE_K // ROW_BLK), 0, 0)),
        ],
        out_shape=[
            jax.ShapeDtypeStruct((D_FF_K, D_MODEL_K), jnp.bfloat16),
            jax.ShapeDtypeStruct((NUM_TILES_K, 1, D_MODEL_K), jnp.float32),
        ],
    )(up_w)
    s = s.reshape(NUM_TILES_K, D_MODEL_K)

    downsign = pl.pallas_call(
        _prep_down_body,
        grid=(D_FF_K // ROW_BLK,),
        in_specs=[pl.BlockSpec((D_MODEL_K, ROW_BLK), lambda g: (0, g))],
        out_specs=pl.BlockSpec((D_MODEL_K, ROW_BLK), lambda g: (0, g)),
        out_shape=jax.ShapeDtypeStruct((D_MODEL_K, D_FF_K), jnp.bfloat16),
    )(down_w)

    winner, gate, xp = pl.pallas_call(
        _router_body,
        grid=(n // TB,),
        in_specs=[
            pl.BlockSpec((TB, D_MODEL_K), lambda g: (g, 0)),
            pl.BlockSpec((NUM_TILES_K, D_MODEL_K), lambda g: (0, 0)),
        ],
        out_specs=[
            pl.BlockSpec((TB,), lambda g: (g,)),
            pl.BlockSpec((TB, NUM_TILES_K), lambda g: (g, 0)),
            pl.BlockSpec((TB, D_MODEL_K // 2), lambda g: (g, 0)),
        ],
        out_shape=[
            jax.ShapeDtypeStruct((n,), jnp.int32),
            jax.ShapeDtypeStruct((n, NUM_TILES_K), jnp.float32),
            jax.ShapeDtypeStruct((n, D_MODEL_K // 2), jnp.int32),
        ],
    )(xf, s)

    dest, btile = pl.pallas_call(
        _dispatch_body,
        out_shape=[
            jax.ShapeDtypeStruct((n // 128, 128), jnp.int32),
            jax.ShapeDtypeStruct((1, G_K), jnp.int32),
        ],
    )(winner.reshape(n // 128, 128))
    dest = dest.reshape(n)
    btile = btile.reshape(G_K)

    sc_scatter, sc_gather = _sc_kernels()
    xbuf = sc_scatter(xp, dest)

    ybuf = pl.pallas_call(
        _ffn_body,
        grid_spec=pltpu.PrefetchScalarGridSpec(
            num_scalar_prefetch=1,
            grid=(G_K,),
            in_specs=[
                pl.BlockSpec((TBM, D_MODEL_K // 2), lambda g, bt: (g, 0)),
                pl.BlockSpec((TILE_K, D_MODEL_K), lambda g, bt: (bt[g], 0)),
                pl.BlockSpec((D_MODEL_K, TILE_K), lambda g, bt: (0, bt[g])),
                pl.BlockSpec((1, TILE_K), lambda g, bt: (0, bt[g])),
                pl.BlockSpec((1, D_MODEL_K), lambda g, bt: (0, 0)),
            ],
            out_specs=pl.BlockSpec((TBM, D_MODEL_K), lambda g, bt: (g, 0)),
        ),
        out_shape=jax.ShapeDtypeStruct((G_K * TBM, D_MODEL_K), jnp.float32),
    )(btile, xbuf, upsign, downsign, up_scales.reshape(1, D_FF_K),
      down_scales.reshape(1, D_MODEL_K))

    out = sc_gather(ybuf, dest)

    return out.reshape(b, t, c), gate.reshape(b, t, NUM_TILES_K)


def kernel(x, up_w, up_scales, down_w, down_scales):
    return _run(x, up_w, up_scales, down_w, down_scales)


# speedup vs baseline: 4.8458x; 1.0053x over previous
import functools

import jax
import jax.numpy as jnp
from jax import lax
from jax.experimental import pallas as pl
from jax.experimental.pallas import tpu as pltpu
from jax.experimental.pallas import tpu_sc as plsc

D_MODEL_K = 2048
NUM_TILES_K = 4
D_FF_K = D_MODEL_K * 4
TILE_K = D_FF_K // NUM_TILES_K
N_K = 2 * 4096
ROW_BLK = 512
TB = 512
TBM = 512
G_K = N_K // TBM + NUM_TILES_K - 1
NC, NS = 2, 16
NW = NC * NS
TPW = N_K // NW
CH = 32


def _prep_up_body(up_ref, upsign_ref, s_ref):
    g = pl.program_id(0)
    usign = jnp.sign(up_ref[...])
    upsign_ref[...] = usign.astype(jnp.bfloat16)
    blk_sum = jnp.sum(usign, axis=0, keepdims=True)[None]
    @pl.when(g % (TILE_K // ROW_BLK) == 0)
    def _init():
        s_ref[...] = jnp.zeros_like(s_ref)
    s_ref[...] += blk_sum


def _prep_down_body(down_ref, downsign_ref):
    downsign_ref[...] = jnp.sign(down_ref[...]).astype(jnp.bfloat16)


def _pack_bf16_pair(lo16, hi16):
    lo_u = lax.convert_element_type(
        lax.bitcast_convert_type(lo16, jnp.uint16), jnp.uint32)
    hi_u = lax.convert_element_type(
        lax.bitcast_convert_type(hi16, jnp.uint16), jnp.uint32)
    return lax.bitcast_convert_type(lo_u | (hi_u << 16), jnp.int32)


def _unpack_bf16_pair(packed):
    pu = lax.bitcast_convert_type(packed, jnp.uint32)
    lo = lax.bitcast_convert_type(
        lax.convert_element_type(pu & 0xFFFF, jnp.uint16), jnp.bfloat16)
    hi = lax.bitcast_convert_type(
        lax.convert_element_type(pu >> 16, jnp.uint16), jnp.bfloat16)
    return lo, hi


def _router_body(x_ref, s_ref, winner_ref, gate_ref, xp_ref):
    xb = x_ref[...]
    m = s_ref[...] * (1.0 / TILE_K)
    denom = jnp.sqrt(jnp.sum(m * m, axis=-1, keepdims=True)) + 1e-8
    sigs = m / denom
    xb16 = xb.astype(jnp.bfloat16)
    xp_ref[...] = _pack_bf16_pair(xb16[:, :D_MODEL_K // 2],
                                  xb16[:, D_MODEL_K // 2:])
    scores = lax.dot_general(xb16, sigs.astype(jnp.bfloat16),
                             (((1,), (1,)), ((), ())),
                             preferred_element_type=jnp.float32)
    winner = jnp.argmax(scores, axis=-1).astype(jnp.int32)
    gate_ref[...] = (
        winner[:, None] == lax.broadcasted_iota(jnp.int32, (1, NUM_TILES_K), 1)
    ).astype(jnp.float32)
    winner_ref[...] = winner


def _cumsum_axis(x, axis):
    size = x.shape[axis]
    sh = 1
    while sh < size:
        if axis == 1:
            shifted = jnp.concatenate(
                [jnp.zeros((x.shape[0], sh), x.dtype), x[:, :-sh]], axis=1)
        else:
            shifted = jnp.concatenate(
                [jnp.zeros((sh, x.shape[1]), x.dtype), x[:-sh, :]], axis=0)
        x = x + shifted
        sh *= 2
    return x


def _dispatch_body(w_ref, dest_ref, btile_ref):
    w = w_ref[...]
    within = jnp.zeros_like(w)
    counts = []
    for t in range(NUM_TILES_K):
        mt = (w == t).astype(jnp.int32)
        lane_cum = _cumsum_axis(mt, axis=1)
        row_tot = lane_cum[:, -1:]
        row_cum = _cumsum_axis(row_tot, axis=0)
        excl = (row_cum - row_tot) + (lane_cum - mt)
        within = within + mt * excl
        counts.append(row_cum[-1:, :])
    nb = [(c + (TBM - 1)) // TBM for c in counts]
    seg = jnp.zeros_like(counts[0])
    dest = within
    cum = jnp.zeros_like(counts[0])
    iota_g = lax.broadcasted_iota(jnp.int32, (1, G_K), 1)
    btile = jnp.zeros((1, G_K), dtype=jnp.int32)
    for t in range(NUM_TILES_K):
        dest = dest + (w == t).astype(jnp.int32) * (seg * TBM)
        cum = cum + nb[t]
        if t < NUM_TILES_K - 1:
            btile = btile + (iota_g >= cum).astype(jnp.int32)
        seg = cum
    dest_ref[...] = dest
    btile_ref[...] = btile


def _sc_scatter_body(x_hbm, dest_hbm, xbuf_hbm, idx_v, rows_v, sem):
    wid = lax.axis_index("s") * NC + lax.axis_index("c")
    base = wid * TPW
    for ci in range(TPW // CH):
        off = base + ci * CH
        pltpu.sync_copy(dest_hbm.at[pl.ds(off, CH)], idx_v)
        pltpu.sync_copy(x_hbm.at[pl.ds(off, CH)], rows_v)
        pltpu.async_copy(rows_v, xbuf_hbm.at[idx_v], sem).wait()


def _sc_gather_body(ybuf_hbm, dest_hbm, out_hbm, idx_v, rows_v, sem):
    wid = lax.axis_index("s") * NC + lax.axis_index("c")
    base = wid * TPW
    for ci in range(TPW // CH):
        off = base + ci * CH
        pltpu.sync_copy(dest_hbm.at[pl.ds(off, CH)], idx_v)
        pltpu.async_copy(ybuf_hbm.at[idx_v], rows_v, sem).wait()
        pltpu.sync_copy(rows_v, out_hbm.at[pl.ds(off, CH)])


@functools.lru_cache(maxsize=None)
def _sc_kernels():
    mesh = plsc.VectorSubcoreMesh(core_axis_name="c", subcore_axis_name="s",
                                  num_cores=NC, num_subcores=NS)
    scatter = pl.kernel(
        _sc_scatter_body,
        out_type=jax.ShapeDtypeStruct((G_K * TBM, D_MODEL_K // 2), jnp.int32),
        mesh=mesh, scratch_types=[
            pltpu.VMEM((CH,), jnp.int32),
            pltpu.VMEM((CH, D_MODEL_K // 2), jnp.int32),
            pltpu.SemaphoreType.DMA,
        ])
    gather = pl.kernel(
        _sc_gather_body,
        out_type=jax.ShapeDtypeStruct((N_K, D_MODEL_K), jnp.float32),
        mesh=mesh, scratch_types=[
            pltpu.VMEM((CH,), jnp.int32),
            pltpu.VMEM((CH, D_MODEL_K), jnp.float32),
            pltpu.SemaphoreType.DMA,
        ])
    return scatter, gather


def _ffn_body(btile_ref, xbuf_ref, upsign_ref, downsign_ref, upsc_ref,
              downsc_ref, ybuf_ref):
    lo, hi = _unpack_bf16_pair(xbuf_ref[...])
    xb16 = jnp.concatenate([lo, hi], axis=1)
    h = lax.dot_general(xb16, upsign_ref[...], (((1,), (1,)), ((), ())),
                        preferred_element_type=jnp.float32)
    h = jnp.maximum(h * upsc_ref[...], 0.0)
    y = lax.dot_general(h.astype(jnp.bfloat16), downsign_ref[...],
                        (((1,), (1,)), ((), ())),
                        preferred_element_type=jnp.float32)
    ybuf_ref[...] = y * downsc_ref[...]


@jax.jit
def _run(x, up_w, up_scales, down_w, down_scales):
    b, t, c = x.shape
    n = b * t
    xf = x.reshape(n, c)

    upsign, s = pl.pallas_call(
        _prep_up_body,
        grid=(D_FF_K // ROW_BLK,),
        in_specs=[
            pl.BlockSpec((ROW_BLK, D_MODEL_K), lambda g: (g, 0)),
        ],
        out_specs=[
            pl.BlockSpec((ROW_BLK, D_MODEL_K), lambda g: (g, 0)),
            pl.BlockSpec((1, 1, D_MODEL_K), lambda g: (g // (TILE_K // ROW_BLK), 0, 0)),
        ],
        out_shape=[
            jax.ShapeDtypeStruct((D_FF_K, D_MODEL_K), jnp.bfloat16),
            jax.ShapeDtypeStruct((NUM_TILES_K, 1, D_MODEL_K), jnp.float32),
        ],
    )(up_w)
    s = s.reshape(NUM_TILES_K, D_MODEL_K)

    downsign = pl.pallas_call(
        _prep_down_body,
        grid=(D_FF_K // ROW_BLK,),
        in_specs=[pl.BlockSpec((D_MODEL_K, ROW_BLK), lambda g: (0, g))],
        out_specs=pl.BlockSpec((D_MODEL_K, ROW_BLK), lambda g: (0, g)),
        out_shape=jax.ShapeDtypeStruct((D_MODEL_K, D_FF_K), jnp.bfloat16),
    )(down_w)

    winner, gate, xp = pl.pallas_call(
        _router_body,
        grid=(n // TB,),
        in_specs=[
            pl.BlockSpec((TB, D_MODEL_K), lambda g: (g, 0)),
            pl.BlockSpec((NUM_TILES_K, D_MODEL_K), lambda g: (0, 0)),
        ],
        out_specs=[
            pl.BlockSpec((TB,), lambda g: (g,)),
            pl.BlockSpec((TB, NUM_TILES_K), lambda g: (g, 0)),
            pl.BlockSpec((TB, D_MODEL_K // 2), lambda g: (g, 0)),
        ],
        out_shape=[
            jax.ShapeDtypeStruct((n,), jnp.int32),
            jax.ShapeDtypeStruct((n, NUM_TILES_K), jnp.float32),
            jax.ShapeDtypeStruct((n, D_MODEL_K // 2), jnp.int32),
        ],
    )(xf, s)

    dest, btile = pl.pallas_call(
        _dispatch_body,
        out_shape=[
            jax.ShapeDtypeStruct((n // 128, 128), jnp.int32),
            jax.ShapeDtypeStruct((1, G_K), jnp.int32),
        ],
    )(winner.reshape(n // 128, 128))
    dest = dest.reshape(n)
    btile = btile.reshape(G_K)

    sc_scatter, sc_gather = _sc_kernels()
    xbuf = sc_scatter(xp, dest)

    ybuf = pl.pallas_call(
        _ffn_body,
        grid_spec=pltpu.PrefetchScalarGridSpec(
            num_scalar_prefetch=1,
            grid=(G_K,),
            in_specs=[
                pl.BlockSpec((TBM, D_MODEL_K // 2), lambda g, bt: (g, 0)),
                pl.BlockSpec((TILE_K, D_MODEL_K), lambda g, bt: (bt[g], 0)),
                pl.BlockSpec((D_MODEL_K, TILE_K), lambda g, bt: (0, bt[g])),
                pl.BlockSpec((1, TILE_K), lambda g, bt: (0, bt[g])),
                pl.BlockSpec((1, D_MODEL_K), lambda g, bt: (0, 0)),
            ],
            out_specs=pl.BlockSpec((TBM, D_MODEL_K), lambda g, bt: (g, 0)),
        ),
        out_shape=jax.ShapeDtypeStruct((G_K * TBM, D_MODEL_K), jnp.float32),
    )(btile, xbuf, upsign, downsign, up_scales.reshape(1, D_FF_K),
      down_scales.reshape(1, D_MODEL_K))

    out = sc_gather(ybuf, dest)

    return out.reshape(b, t, c), gate.reshape(b, t, NUM_TILES_K)


def kernel(x, up_w, up_scales, down_w, down_scales):
    return _run(x, up_w, up_scales, down_w, down_scales)


# speedup vs baseline: 4.8859x; 1.0083x over previous
import functools

import jax
import jax.numpy as jnp
from jax import lax
from jax.experimental import pallas as pl
from jax.experimental.pallas import tpu as pltpu
from jax.experimental.pallas import tpu_sc as plsc

D_MODEL_K = 2048
NUM_TILES_K = 4
D_FF_K = D_MODEL_K * 4
TILE_K = D_FF_K // NUM_TILES_K
N_K = 2 * 4096
ROW_BLK = 512
TB = 512
TBM = 512
G_K = N_K // TBM + NUM_TILES_K - 1
NC, NS = 2, 16
NW = NC * NS
TPW = N_K // NW
CH = 32
CHG = 16


def _prep_body(up_ref, down_ref, upsign_ref, downsign_ref, s_ref):
    g = pl.program_id(0)
    usign = jnp.sign(up_ref[...])
    upsign_ref[...] = usign.astype(jnp.bfloat16)
    downsign_ref[...] = jnp.sign(down_ref[...]).astype(jnp.bfloat16)
    blk_sum = jnp.sum(usign, axis=0, keepdims=True)[None]
    @pl.when(g % (TILE_K // ROW_BLK) == 0)
    def _init():
        s_ref[...] = jnp.zeros_like(s_ref)
    s_ref[...] += blk_sum


def _pack_bf16_pair(lo16, hi16):
    lo_u = lax.convert_element_type(
        lax.bitcast_convert_type(lo16, jnp.uint16), jnp.uint32)
    hi_u = lax.convert_element_type(
        lax.bitcast_convert_type(hi16, jnp.uint16), jnp.uint32)
    return lax.bitcast_convert_type(lo_u | (hi_u << 16), jnp.int32)


def _unpack_bf16_pair(packed):
    pu = lax.bitcast_convert_type(packed, jnp.uint32)
    lo = lax.bitcast_convert_type(
        lax.convert_element_type(pu & 0xFFFF, jnp.uint16), jnp.bfloat16)
    hi = lax.bitcast_convert_type(
        lax.convert_element_type(pu >> 16, jnp.uint16), jnp.bfloat16)
    return lo, hi


def _router_body(x_ref, s_ref, winner_ref, gate_ref, xp_ref):
    xb = x_ref[...]
    m = s_ref[...] * (1.0 / TILE_K)
    denom = jnp.sqrt(jnp.sum(m * m, axis=-1, keepdims=True)) + 1e-8
    sigs = m / denom
    xb16 = xb.astype(jnp.bfloat16)
    xp_ref[...] = _pack_bf16_pair(xb16[:, :D_MODEL_K // 2],
                                  xb16[:, D_MODEL_K // 2:])
    scores = lax.dot_general(xb16, sigs.astype(jnp.bfloat16),
                             (((1,), (1,)), ((), ())),
                             preferred_element_type=jnp.float32)
    winner = jnp.argmax(scores, axis=-1).astype(jnp.int32)
    gate_ref[...] = (
        winner[:, None] == lax.broadcasted_iota(jnp.int32, (1, NUM_TILES_K), 1)
    ).astype(jnp.float32)
    winner_ref[...] = winner


def _cumsum_axis(x, axis):
    size = x.shape[axis]
    sh = 1
    while sh < size:
        if axis == 1:
            shifted = jnp.concatenate(
                [jnp.zeros((x.shape[0], sh), x.dtype), x[:, :-sh]], axis=1)
        else:
            shifted = jnp.concatenate(
                [jnp.zeros((sh, x.shape[1]), x.dtype), x[:-sh, :]], axis=0)
        x = x + shifted
        sh *= 2
    return x


def _dispatch_body(w_ref, dest_ref, btile_ref, nused_ref):
    w = w_ref[...]
    within = jnp.zeros_like(w)
    counts = []
    for t in range(NUM_TILES_K):
        mt = (w == t).astype(jnp.int32)
        lane_cum = _cumsum_axis(mt, axis=1)
        row_tot = lane_cum[:, -1:]
        row_cum = _cumsum_axis(row_tot, axis=0)
        excl = (row_cum - row_tot) + (lane_cum - mt)
        within = within + mt * excl
        counts.append(row_cum[-1:, :])
    nb = [(c + (TBM - 1)) // TBM for c in counts]
    seg = jnp.zeros_like(counts[0])
    dest = within
    cum = jnp.zeros_like(counts[0])
    iota_g = lax.broadcasted_iota(jnp.int32, (1, G_K), 1)
    btile = jnp.zeros((1, G_K), dtype=jnp.int32)
    for t in range(NUM_TILES_K):
        dest = dest + (w == t).astype(jnp.int32) * (seg * TBM)
        cum = cum + nb[t]
        if t < NUM_TILES_K - 1:
            btile = btile + (iota_g >= cum).astype(jnp.int32)
        seg = cum
    dest_ref[...] = dest
    btile_ref[...] = btile
    nused_ref[...] = cum


def _sc_scatter_body(x_hbm, dest_hbm, xbuf_hbm,
                     idx0, idx1, row0, row1, si0, si1, so0, so1):
    wid = lax.axis_index("s") * NC + lax.axis_index("c")
    base = wid * TPW
    idx, row, si, so = [idx0, idx1], [row0, row1], [si0, si1], [so0, so1]
    nchunk = TPW // CH

    def start_in(ci):
        b = ci & 1
        off = base + ci * CH
        d1 = pltpu.async_copy(dest_hbm.at[pl.ds(off, CH)], idx[b], si[b])
        d2 = pltpu.async_copy(x_hbm.at[pl.ds(off, CH)], row[b], si[b])
        return (d1, d2)

    ins = {0: start_in(0)}
    outs = {}
    for ci in range(nchunk):
        b = ci & 1
        for d in ins.pop(ci):
            d.wait()
        outs[ci] = pltpu.async_copy(row[b], xbuf_hbm.at[idx[b]], so[b])
        if ci + 1 < nchunk:
            if ci - 1 >= 0:
                outs.pop(ci - 1).wait()
            ins[ci + 1] = start_in(ci + 1)
    outs.pop(nchunk - 1).wait()


def _sc_gather_body(ybuf_hbm, dest_hbm, out_hbm,
                    idx0, idx1, row0, row1, si0, si1, so0, so1):
    wid = lax.axis_index("s") * NC + lax.axis_index("c")
    base = wid * TPW
    idx, row, si, so = [idx0, idx1], [row0, row1], [si0, si1], [so0, so1]
    nchunk = TPW // CHG

    def start_in(ci):
        b = ci & 1
        off = base + ci * CHG
        pltpu.sync_copy(dest_hbm.at[pl.ds(off, CHG)], idx[b])
        return pltpu.async_copy(ybuf_hbm.at[idx[b]], row[b], si[b])

    ins = {0: start_in(0)}
    outs = {}
    for ci in range(nchunk):
        b = ci & 1
        ins.pop(ci).wait()
        off = base + ci * CHG
        outs[ci] = pltpu.async_copy(row[b], out_hbm.at[pl.ds(off, CHG)], so[b])
        if ci + 1 < nchunk:
            if ci - 1 >= 0:
                outs.pop(ci - 1).wait()
            ins[ci + 1] = start_in(ci + 1)
    outs.pop(nchunk - 1).wait()


@functools.lru_cache(maxsize=None)
def _sc_kernels():
    mesh = plsc.VectorSubcoreMesh(core_axis_name="c", subcore_axis_name="s",
                                  num_cores=NC, num_subcores=NS)
    scatter = pl.kernel(
        _sc_scatter_body,
        out_type=jax.ShapeDtypeStruct((G_K * TBM, D_MODEL_K // 2), jnp.int32),
        mesh=mesh, scratch_types=[
            pltpu.VMEM((CH,), jnp.int32),
            pltpu.VMEM((CH,), jnp.int32),
            pltpu.VMEM((CH, D_MODEL_K // 2), jnp.int32),
            pltpu.VMEM((CH, D_MODEL_K // 2), jnp.int32),
            pltpu.SemaphoreType.DMA,
            pltpu.SemaphoreType.DMA,
            pltpu.SemaphoreType.DMA,
            pltpu.SemaphoreType.DMA,
        ])
    gather = pl.kernel(
        _sc_gather_body,
        out_type=jax.ShapeDtypeStruct((N_K, D_MODEL_K), jnp.float32),
        mesh=mesh, scratch_types=[
            pltpu.VMEM((CHG,), jnp.int32),
            pltpu.VMEM((CHG,), jnp.int32),
            pltpu.VMEM((CHG, D_MODEL_K), jnp.float32),
            pltpu.VMEM((CHG, D_MODEL_K), jnp.float32),
            pltpu.SemaphoreType.DMA,
            pltpu.SemaphoreType.DMA,
            pltpu.SemaphoreType.DMA,
            pltpu.SemaphoreType.DMA,
        ])
    return scatter, gather


def _ffn_body(btile_ref, nused_ref, xbuf_ref, upsign_ref, downsign_ref,
              upsc_ref, downsc_ref, ybuf_ref):
    @pl.when(pl.program_id(0) < nused_ref[0])
    def _compute():
        lo, hi = _unpack_bf16_pair(xbuf_ref[...])
        xb16 = jnp.concatenate([lo, hi], axis=1)
        h = lax.dot_general(xb16, upsign_ref[...], (((1,), (1,)), ((), ())),
                            preferred_element_type=jnp.float32)
        h = jnp.maximum(h * upsc_ref[...], 0.0)
        y = lax.dot_general(h.astype(jnp.bfloat16), downsign_ref[...],
                            (((1,), (1,)), ((), ())),
                            preferred_element_type=jnp.float32)
        ybuf_ref[...] = y * downsc_ref[...]


@jax.jit
def _run(x, up_w, up_scales, down_w, down_scales):
    b, t, c = x.shape
    n = b * t
    xf = x.reshape(n, c)

    upsign, downsign, s = pl.pallas_call(
        _prep_body,
        grid=(D_FF_K // ROW_BLK,),
        in_specs=[
            pl.BlockSpec((ROW_BLK, D_MODEL_K), lambda g: (g, 0)),
            pl.BlockSpec((D_MODEL_K, ROW_BLK), lambda g: (0, g)),
        ],
        out_specs=[
            pl.BlockSpec((ROW_BLK, D_MODEL_K), lambda g: (g, 0)),
            pl.BlockSpec((D_MODEL_K, ROW_BLK), lambda g: (0, g)),
            pl.BlockSpec((1, 1, D_MODEL_K), lambda g: (g // (TILE_K // ROW_BLK), 0, 0)),
        ],
        out_shape=[
            jax.ShapeDtypeStruct((D_FF_K, D_MODEL_K), jnp.bfloat16),
            jax.ShapeDtypeStruct((D_MODEL_K, D_FF_K), jnp.bfloat16),
            jax.ShapeDtypeStruct((NUM_TILES_K, 1, D_MODEL_K), jnp.float32),
        ],
    )(up_w, down_w)
    s = s.reshape(NUM_TILES_K, D_MODEL_K)

    winner, gate, xp = pl.pallas_call(
        _router_body,
        grid=(n // TB,),
        in_specs=[
            pl.BlockSpec((TB, D_MODEL_K), lambda g: (g, 0)),
            pl.BlockSpec((NUM_TILES_K, D_MODEL_K), lambda g: (0, 0)),
        ],
        out_specs=[
            pl.BlockSpec((TB,), lambda g: (g,)),
            pl.BlockSpec((TB, NUM_TILES_K), lambda g: (g, 0)),
            pl.BlockSpec((TB, D_MODEL_K // 2), lambda g: (g, 0)),
        ],
        out_shape=[
            jax.ShapeDtypeStruct((n,), jnp.int32),
            jax.ShapeDtypeStruct((n, NUM_TILES_K), jnp.float32),
            jax.ShapeDtypeStruct((n, D_MODEL_K // 2), jnp.int32),
        ],
    )(xf, s)

    dest, btile, nused = pl.pallas_call(
        _dispatch_body,
        out_shape=[
            jax.ShapeDtypeStruct((n // 128, 128), jnp.int32),
            jax.ShapeDtypeStruct((1, G_K), jnp.int32),
            jax.ShapeDtypeStruct((1, 1), jnp.int32),
        ],
    )(winner.reshape(n // 128, 128))
    dest = dest.reshape(n)
    btile = btile.reshape(G_K)
    nused = nused.reshape(1)

    sc_scatter, sc_gather = _sc_kernels()
    xbuf = sc_scatter(xp, dest)

    ybuf = pl.pallas_call(
        _ffn_body,
        grid_spec=pltpu.PrefetchScalarGridSpec(
            num_scalar_prefetch=2,
            grid=(G_K,),
            in_specs=[
                pl.BlockSpec((TBM, D_MODEL_K // 2), lambda g, bt, nu: (g, 0)),
                pl.BlockSpec((TILE_K, D_MODEL_K), lambda g, bt, nu: (bt[g], 0)),
                pl.BlockSpec((D_MODEL_K, TILE_K), lambda g, bt, nu: (0, bt[g])),
                pl.BlockSpec((1, TILE_K), lambda g, bt, nu: (0, bt[g])),
                pl.BlockSpec((1, D_MODEL_K), lambda g, bt, nu: (0, 0)),
            ],
            out_specs=pl.BlockSpec((TBM, D_MODEL_K), lambda g, bt, nu: (g, 0)),
        ),
        out_shape=jax.ShapeDtypeStruct((G_K * TBM, D_MODEL_K), jnp.float32),
    )(btile, nused, xbuf, upsign, downsign, up_scales.reshape(1, D_FF_K),
      down_scales.reshape(1, D_MODEL_K))

    out = sc_gather(ybuf, dest)

    return out.reshape(b, t, c), gate.reshape(b, t, NUM_TILES_K)


def kernel(x, up_w, up_scales, down_w, down_scales):
    return _run(x, up_w, up_scales, down_w, down_scales)


# speedup vs baseline: 4.9742x; 1.0181x over previous
import functools

import jax
import jax.numpy as jnp
from jax import lax
from jax.experimental import pallas as pl
from jax.experimental.pallas import tpu as pltpu
from jax.experimental.pallas import tpu_sc as plsc

D_MODEL_K = 2048
NUM_TILES_K = 4
D_FF_K = D_MODEL_K * 4
TILE_K = D_FF_K // NUM_TILES_K
N_K = 2 * 4096
ROW_BLK = 512
TB = 512
TBM = 512
G_K = N_K // TBM + NUM_TILES_K - 1
NC, NS = 2, 16
NW = NC * NS
TPW = N_K // NW
CH = 32
CHG = 16


N_PREP = D_FF_K // ROW_BLK
N_RTR = N_K // TB


def _fused_body(up_ref, down_ref, x_ref,
                upsign_ref, downsign_ref, gate_ref, xp_ref,
                dest_ref, btile_ref, nused_ref, s_sc, w_sc):
    g = pl.program_id(0)

    @pl.when(g < N_PREP)
    def _prep():
        usign = jnp.sign(up_ref[...])
        upsign_ref[...] = usign.astype(jnp.bfloat16)
        downsign_ref[...] = jnp.sign(down_ref[...]).astype(jnp.bfloat16)
        blk_sum = jnp.sum(usign, axis=0, keepdims=True)
        tile = g // (TILE_K // ROW_BLK)
        row_is_tile = (lax.broadcasted_iota(jnp.int32, (8, 1), 0) == tile
                       ).astype(jnp.float32)
        @pl.when(g == 0)
        def _init():
            s_sc[...] = jnp.zeros_like(s_sc)
        s_sc[...] += row_is_tile * blk_sum

    @pl.when((g >= N_PREP) & (g < N_PREP + N_RTR))
    def _router():
        xb = x_ref[...]
        m = s_sc[...][:NUM_TILES_K] * (1.0 / TILE_K)
        denom = jnp.sqrt(jnp.sum(m * m, axis=-1, keepdims=True)) + 1e-8
        sigs = m / denom
        xb16 = xb.astype(jnp.bfloat16)
        xp_ref[...] = _pack_bf16_pair(xb16[:, :D_MODEL_K // 2],
                                      xb16[:, D_MODEL_K // 2:])
        scores = lax.dot_general(xb16, sigs.astype(jnp.bfloat16),
                                 (((1,), (1,)), ((), ())),
                                 preferred_element_type=jnp.float32)
        winner = jnp.argmax(scores, axis=-1).astype(jnp.int32)
        gate_ref[...] = (
            winner[:, None] == lax.broadcasted_iota(jnp.int32, (1, NUM_TILES_K), 1)
        ).astype(jnp.float32)
        w_sc[pl.ds((g - N_PREP) * (TB // 128), TB // 128), :] = (
            winner.reshape(TB // 128, 128))

    @pl.when(g == N_PREP + N_RTR)
    def _dispatch():
        _dispatch_math(w_sc[...], dest_ref, btile_ref, nused_ref)


def _pack_bf16_pair(lo16, hi16):
    lo_u = lax.convert_element_type(
        lax.bitcast_convert_type(lo16, jnp.uint16), jnp.uint32)
    hi_u = lax.convert_element_type(
        lax.bitcast_convert_type(hi16, jnp.uint16), jnp.uint32)
    return lax.bitcast_convert_type(lo_u | (hi_u << 16), jnp.int32)


def _unpack_bf16_pair(packed):
    pu = lax.bitcast_convert_type(packed, jnp.uint32)
    lo = lax.bitcast_convert_type(
        lax.convert_element_type(pu & 0xFFFF, jnp.uint16), jnp.bfloat16)
    hi = lax.bitcast_convert_type(
        lax.convert_element_type(pu >> 16, jnp.uint16), jnp.bfloat16)
    return lo, hi


def _cumsum_axis(x, axis):
    size = x.shape[axis]
    sh = 1
    while sh < size:
        if axis == 1:
            shifted = jnp.concatenate(
                [jnp.zeros((x.shape[0], sh), x.dtype), x[:, :-sh]], axis=1)
        else:
            shifted = jnp.concatenate(
                [jnp.zeros((sh, x.shape[1]), x.dtype), x[:-sh, :]], axis=0)
        x = x + shifted
        sh *= 2
    return x


def _dispatch_math(w, dest_ref, btile_ref, nused_ref):
    within = jnp.zeros_like(w)
    counts = []
    for t in range(NUM_TILES_K):
        mt = (w == t).astype(jnp.int32)
        lane_cum = _cumsum_axis(mt, axis=1)
        row_tot = lane_cum[:, -1:]
        row_cum = _cumsum_axis(row_tot, axis=0)
        excl = (row_cum - row_tot) + (lane_cum - mt)
        within = within + mt * excl
        counts.append(row_cum[-1:, :])
    nb = [(c + (TBM - 1)) // TBM for c in counts]
    seg = jnp.zeros_like(counts[0])
    dest = within
    cum = jnp.zeros_like(counts[0])
    iota_g = lax.broadcasted_iota(jnp.int32, (1, G_K), 1)
    btile = jnp.zeros((1, G_K), dtype=jnp.int32)
    for t in range(NUM_TILES_K):
        dest = dest + (w == t).astype(jnp.int32) * (seg * TBM)
        cum = cum + nb[t]
        if t < NUM_TILES_K - 1:
            btile = btile + (iota_g >= cum).astype(jnp.int32)
        seg = cum
    dest_ref[...] = dest
    btile_ref[...] = btile
    nused_ref[...] = cum


def _sc_scatter_body(x_hbm, dest_hbm, xbuf_hbm,
                     idx0, idx1, row0, row1, si0, si1, so0, so1):
    wid = lax.axis_index("s") * NC + lax.axis_index("c")
    base = wid * TPW
    idx, row, si, so = [idx0, idx1], [row0, row1], [si0, si1], [so0, so1]
    nchunk = TPW // CH

    def start_in(ci):
        b = ci & 1
        off = base + ci * CH
        d1 = pltpu.async_copy(dest_hbm.at[pl.ds(off, CH)], idx[b], si[b])
        d2 = pltpu.async_copy(x_hbm.at[pl.ds(off, CH)], row[b], si[b])
        return (d1, d2)

    ins = {0: start_in(0)}
    outs = {}
    for ci in range(nchunk):
        b = ci & 1
        for d in ins.pop(ci):
            d.wait()
        outs[ci] = pltpu.async_copy(row[b], xbuf_hbm.at[idx[b]], so[b])
        if ci + 1 < nchunk:
            if ci - 1 >= 0:
                outs.pop(ci - 1).wait()
            ins[ci + 1] = start_in(ci + 1)
    outs.pop(nchunk - 1).wait()


def _sc_gather_body(ybuf_hbm, dest_hbm, out_hbm,
                    idx0, idx1, row0, row1, si0, si1, so0, so1):
    wid = lax.axis_index("s") * NC + lax.axis_index("c")
    base = wid * TPW
    idx, row, si, so = [idx0, idx1], [row0, row1], [si0, si1], [so0, so1]
    nchunk = TPW // CHG

    def start_in(ci):
        b = ci & 1
        off = base + ci * CHG
        pltpu.sync_copy(dest_hbm.at[pl.ds(off, CHG)], idx[b])
        return pltpu.async_copy(ybuf_hbm.at[idx[b]], row[b], si[b])

    ins = {0: start_in(0)}
    outs = {}
    for ci in range(nchunk):
        b = ci & 1
        ins.pop(ci).wait()
        off = base + ci * CHG
        outs[ci] = pltpu.async_copy(row[b], out_hbm.at[pl.ds(off, CHG)], so[b])
        if ci + 1 < nchunk:
            if ci - 1 >= 0:
                outs.pop(ci - 1).wait()
            ins[ci + 1] = start_in(ci + 1)
    outs.pop(nchunk - 1).wait()


@functools.lru_cache(maxsize=None)
def _sc_kernels():
    mesh = plsc.VectorSubcoreMesh(core_axis_name="c", subcore_axis_name="s",
                                  num_cores=NC, num_subcores=NS)
    scatter = pl.kernel(
        _sc_scatter_body,
        out_type=jax.ShapeDtypeStruct((G_K * TBM, D_MODEL_K // 2), jnp.int32),
        mesh=mesh, scratch_types=[
            pltpu.VMEM((CH,), jnp.int32),
            pltpu.VMEM((CH,), jnp.int32),
            pltpu.VMEM((CH, D_MODEL_K // 2), jnp.int32),
            pltpu.VMEM((CH, D_MODEL_K // 2), jnp.int32),
            pltpu.SemaphoreType.DMA,
            pltpu.SemaphoreType.DMA,
            pltpu.SemaphoreType.DMA,
            pltpu.SemaphoreType.DMA,
        ])
    gather = pl.kernel(
        _sc_gather_body,
        out_type=jax.ShapeDtypeStruct((N_K, D_MODEL_K), jnp.float32),
        mesh=mesh, scratch_types=[
            pltpu.VMEM((CHG,), jnp.int32),
            pltpu.VMEM((CHG,), jnp.int32),
            pltpu.VMEM((CHG, D_MODEL_K), jnp.float32),
            pltpu.VMEM((CHG, D_MODEL_K), jnp.float32),
            pltpu.SemaphoreType.DMA,
            pltpu.SemaphoreType.DMA,
            pltpu.SemaphoreType.DMA,
            pltpu.SemaphoreType.DMA,
        ])
    return scatter, gather


def _ffn_body(btile_ref, nused_ref, xbuf_ref, upsign_ref, downsign_ref,
              upsc_ref, downsc_ref, ybuf_ref):
    @pl.when(pl.program_id(0) < nused_ref[0])
    def _compute():
        lo, hi = _unpack_bf16_pair(xbuf_ref[...])
        xb16 = jnp.concatenate([lo, hi], axis=1)
        h = lax.dot_general(xb16, upsign_ref[...], (((1,), (1,)), ((), ())),
                            preferred_element_type=jnp.float32)
        h = jnp.maximum(h * upsc_ref[...], 0.0)
        y = lax.dot_general(h.astype(jnp.bfloat16), downsign_ref[...],
                            (((1,), (1,)), ((), ())),
                            preferred_element_type=jnp.float32)
        ybuf_ref[...] = y * downsc_ref[...]


@jax.jit
def _run(x, up_w, up_scales, down_w, down_scales):
    b, t, c = x.shape
    n = b * t
    xf = x.reshape(n, c)

    rtr0 = N_PREP

    upsign, downsign, gate, xp, dest, btile, nused = pl.pallas_call(
        _fused_body,
        grid=(N_PREP + N_RTR + 1,),
        in_specs=[
            pl.BlockSpec((ROW_BLK, D_MODEL_K),
                         lambda g: (jnp.minimum(g, N_PREP - 1), 0)),
            pl.BlockSpec((D_MODEL_K, ROW_BLK),
                         lambda g: (0, jnp.minimum(g, N_PREP - 1))),
            pl.BlockSpec((TB, D_MODEL_K),
                         lambda g: (jnp.clip(g - rtr0, 0, N_RTR - 1), 0)),
        ],
        out_specs=[
            pl.BlockSpec((ROW_BLK, D_MODEL_K),
                         lambda g: (jnp.minimum(g, N_PREP - 1), 0)),
            pl.BlockSpec((D_MODEL_K, ROW_BLK),
                         lambda g: (0, jnp.minimum(g, N_PREP - 1))),
            pl.BlockSpec((TB, NUM_TILES_K),
                         lambda g: (jnp.clip(g - rtr0, 0, N_RTR - 1), 0)),
            pl.BlockSpec((TB, D_MODEL_K // 2),
                         lambda g: (jnp.clip(g - rtr0, 0, N_RTR - 1), 0)),
            pl.BlockSpec((N_K // 128, 128), lambda g: (0, 0)),
            pl.BlockSpec((1, G_K), lambda g: (0, 0)),
            pl.BlockSpec((1, 1), lambda g: (0, 0)),
        ],
        out_shape=[
            jax.ShapeDtypeStruct((D_FF_K, D_MODEL_K), jnp.bfloat16),
            jax.ShapeDtypeStruct((D_MODEL_K, D_FF_K), jnp.bfloat16),
            jax.ShapeDtypeStruct((n, NUM_TILES_K), jnp.float32),
            jax.ShapeDtypeStruct((n, D_MODEL_K // 2), jnp.int32),
            jax.ShapeDtypeStruct((n // 128, 128), jnp.int32),
            jax.ShapeDtypeStruct((1, G_K), jnp.int32),
            jax.ShapeDtypeStruct((1, 1), jnp.int32),
        ],
        scratch_shapes=[
            pltpu.VMEM((8, D_MODEL_K), jnp.float32),
            pltpu.VMEM((N_K // 128, 128), jnp.int32),
        ],
    )(up_w, down_w, xf)
    dest = dest.reshape(n)
    btile = btile.reshape(G_K)
    nused = nused.reshape(1)

    sc_scatter, sc_gather = _sc_kernels()
    xbuf = sc_scatter(xp, dest)

    ybuf = pl.pallas_call(
        _ffn_body,
        grid_spec=pltpu.PrefetchScalarGridSpec(
            num_scalar_prefetch=2,
            grid=(G_K,),
            in_specs=[
                pl.BlockSpec((TBM, D_MODEL_K // 2), lambda g, bt, nu: (g, 0)),
                pl.BlockSpec((TILE_K, D_MODEL_K), lambda g, bt, nu: (bt[g], 0)),
                pl.BlockSpec((D_MODEL_K, TILE_K), lambda g, bt, nu: (0, bt[g])),
                pl.BlockSpec((1, TILE_K), lambda g, bt, nu: (0, bt[g])),
                pl.BlockSpec((1, D_MODEL_K), lambda g, bt, nu: (0, 0)),
            ],
            out_specs=pl.BlockSpec((TBM, D_MODEL_K), lambda g, bt, nu: (g, 0)),
        ),
        out_shape=jax.ShapeDtypeStruct((G_K * TBM, D_MODEL_K), jnp.float32),
    )(btile, nused, xbuf, upsign, downsign, up_scales.reshape(1, D_FF_K),
      down_scales.reshape(1, D_MODEL_K))

    out = sc_gather(ybuf, dest)

    return out.reshape(b, t, c), gate.reshape(b, t, NUM_TILES_K)


def kernel(x, up_w, up_scales, down_w, down_scales):
    return _run(x, up_w, up_scales, down_w, down_scales)
